# baseline (speedup 1.0000x reference)
import functools

import jax
import jax.numpy as jnp
from jax import lax
from jax.experimental import pallas as pl
from jax.experimental.pallas import tpu as pltpu
from jax.experimental.pallas import tpu_sc as plsc

K_NB = 16
PQ = 3
BQ = 384
NW = 32
CQ = 48
IN_CH = 32
HID = 64
Q_PAD = 52224
F_ROWS = Q_PAD // PQ
N_CH = Q_PAD // (NW * CQ)


def _sc_chunk(ci, wid, idx_hbm, table_hbm, qc_hbm, rows_hbm, ltc_v, gbuf,
              cur, nxt, sem_g, sem_i, sem_w):
    idx_v, qc_v, rbuf = cur
    idx_n, qc_n, _ = nxt
    qbase = wid * (N_CH * CQ) + ci * CQ
    lanes = lax.broadcasted_iota(jnp.int32, (16,), 0)

    @pl.when(ci + 1 < N_CH)
    def _prefetch():
        nq = qbase + CQ
        pltpu.async_copy(idx_hbm.at[pl.ds(nq * K_NB, CQ * K_NB)], idx_n,
                         sem_i)
        pltpu.async_copy(qc_hbm.at[pl.ds(nq * 2, CQ * 2)], qc_n, sem_i)

    pltpu.make_async_copy(table_hbm.at[pl.ds(0, CQ * K_NB), :], gbuf,
                          sem_g).wait()

    @pl.when(ci >= 2)
    def _wb_wait():
        pltpu.make_async_copy(rows_hbm.at[:, pl.ds(0, CQ // PQ), :], rbuf,
                              sem_w).wait()

    def repack(r, c2):
        def repack_k(k, c3):
            for m in range(6):
                rbuf[k, r, pl.ds(m * 16, 16)] = (
                    gbuf[(PQ * r + m // 2) * K_NB + k,
                         pl.ds((m % 2) * 16, 16)])
            return c3

        return lax.fori_loop(0, K_NB, repack_k, c2)

    lax.fori_loop(0, CQ // PQ, repack, 0)

    @pl.when(ci + 1 < N_CH)
    def _next_gather():
        pltpu.make_async_copy(idx_hbm.at[pl.ds(0, CQ * K_NB)], idx_n,
                              sem_i).wait()
        pltpu.make_async_copy(qc_hbm.at[pl.ds(0, CQ * 2)], qc_n,
                              sem_i).wait()
        pltpu.async_copy(table_hbm.at[idx_n], gbuf, sem_g)

    def regroup(g, c2):
        qloc = g * 16 + lanes
        rrow = (qloc * 21846) >> 16
        lbase = 2 * (qloc - 3 * rrow)
        qcx = plsc.load_gather(qc_v, [qloc * 2])
        qcy = plsc.load_gather(qc_v, [qloc * 2 + 1])

        def regroup_k(k, c3):
            krow = jnp.full((16,), 0, jnp.int32) + k
            vec = plsc.load_gather(idx_v, [qloc * K_NB + k])
            yx = plsc.load_gather(ltc_v, [vec * 2])
            yy = plsc.load_gather(ltc_v, [vec * 2 + 1])
            plsc.store_scatter(rbuf, [krow, rrow, 96 + lbase], yx)
            plsc.store_scatter(rbuf, [krow, rrow, 97 + lbase], yy)
            plsc.store_scatter(rbuf, [krow, rrow, 102 + lbase], qcx)
            plsc.store_scatter(rbuf, [krow, rrow, 103 + lbase], qcy)
            return c3

        return lax.fori_loop(0, K_NB, regroup_k, c2)

    lax.fori_loop(0, CQ // 16, regroup, 0)
    pltpu.async_copy(rbuf, rows_hbm.at[:, pl.ds(qbase // PQ, CQ // PQ), :],
                     sem_w)


def _sc_body(idx_hbm, table_hbm, ltc_hbm, qc_hbm, rows_hbm,
             idx0, qc0, rbuf0, idx1, qc1, rbuf1,
             gbuf, ltc_v, sem_g, sem_i, sem_w):
    cid = lax.axis_index("c")
    sid = lax.axis_index("s")
    wid = sid * 2 + cid
    qbase0 = wid * (N_CH * CQ)
    lanes = lax.broadcasted_iota(jnp.int32, (16,), 0)
    zeros16 = jnp.zeros((16,), jnp.float32)

    pltpu.sync_copy(ltc_hbm, ltc_v)

    def zinit(k, c2):
        krow = jnp.full((16,), 0, jnp.int32) + k

        def zinit_r(r, c3):
            rrow = jnp.full((16,), 0, jnp.int32) + r
            for rbuf in (rbuf0, rbuf1):
                plsc.store_scatter(rbuf, [krow, rrow, 108 + lanes], zeros16)
                plsc.store_scatter(rbuf, [krow, rrow, 112 + lanes], zeros16)
            return c3

        return lax.fori_loop(0, CQ // PQ, zinit_r, c2)

    lax.fori_loop(0, K_NB, zinit, 0)

    pltpu.sync_copy(idx_hbm.at[pl.ds(qbase0 * K_NB, CQ * K_NB)], idx0)
    pltpu.sync_copy(qc_hbm.at[pl.ds(qbase0 * 2, CQ * 2)], qc0)
    pltpu.async_copy(table_hbm.at[idx0], gbuf, sem_g)

    set0 = (idx0, qc0, rbuf0)
    set1 = (idx1, qc1, rbuf1)

    def pair_body(i, carry):
        _sc_chunk(2 * i, wid, idx_hbm, table_hbm, qc_hbm, rows_hbm, ltc_v,
                  gbuf, set0, set1, sem_g, sem_i, sem_w)
        _sc_chunk(2 * i + 1, wid, idx_hbm, table_hbm, qc_hbm, rows_hbm,
                  ltc_v, gbuf, set1, set0, sem_g, sem_i, sem_w)
        return carry

    lax.fori_loop(0, N_CH // 2, pair_body, 0)
    for rbuf in (rbuf0, rbuf1):
        pltpu.make_async_copy(rows_hbm.at[:, pl.ds(0, CQ // PQ), :], rbuf,
                              sem_w).wait()


def _sc_gather(idx_pad, table, ltc_flat, qc_flat):
    mesh = plsc.VectorSubcoreMesh(core_axis_name="c", subcore_axis_name="s")
    dbuf = lambda: [
        pltpu.VMEM((CQ * K_NB,), jnp.int32),
        pltpu.VMEM((CQ * 2,), jnp.float32),
        pltpu.VMEM((K_NB, CQ // PQ, 128), jnp.float32),
    ]
    return pl.kernel(
        _sc_body,
        out_type=jax.ShapeDtypeStruct((K_NB, F_ROWS, 128), jnp.float32),
        mesh=mesh,
        scratch_types=dbuf() + dbuf() + [
            pltpu.VMEM((CQ * K_NB, IN_CH), jnp.float32),
            pltpu.VMEM((ltc_flat.shape[0],), jnp.float32),
            pltpu.SemaphoreType.DMA,
            pltpu.SemaphoreType.DMA,
            pltpu.SemaphoreType.DMA,
        ],
        compiler_params=pltpu.CompilerParams(needs_layout_passes=False,
                                             use_tc_tiling_on_sc=False),
    )(idx_pad, table, ltc_flat, qc_flat)


def _dot(a, b):
    return jax.lax.dot_general(a, b, (((1,), (0,)), ((), ())),
                               preferred_element_type=jnp.float32)


def _tc_body(fr_ref, W0_ref, b0_ref, W1_ref, b1_ref, pW0_ref, pb0_ref,
             pW1_ref, pb1_ref, out_ref):
    nr = BQ // PQ
    fr = fr_ref[...]
    fr2 = fr.reshape(K_NB * nr, 128)
    h = jax.nn.gelu(_dot(fr2, W0_ref[...]) + b0_ref[...])
    kern = (_dot(h, W1_ref[...]) + b1_ref[...]).reshape(K_NB, nr, 128)
    dec = jnp.sum(kern * fr, axis=0)
    h2 = jax.nn.gelu(_dot(dec, pW0_ref[...]) + pb0_ref[...])
    out_ref[...] = _dot(h2, pW1_ref[...]) + pb1_ref[...]


def kernel(latent_tokens_coord, rndata, query_coord, nbr_index, nbr_row_splits,
           kW0, kb0, kW1, kb1, pW0, pb0, pW1, pb1):
    Q = query_coord.shape[0]
    K = nbr_index.shape[0] // Q
    assert K == K_NB
    out_ch = pW1.shape[1]
    proj_ch = pW0.shape[1]
    nb = Q_PAD // BQ

    idx_pad = jnp.pad(nbr_index, (0, (Q_PAD - Q) * K))
    qc_flat = jnp.pad(query_coord, ((0, Q_PAD - Q), (0, 0))).reshape(-1)
    rows = _sc_gather(idx_pad, rndata[0], latent_tokens_coord.reshape(-1),
                      qc_flat)

    W0 = jnp.zeros((128, PQ * HID), jnp.float32)
    W1 = jnp.zeros((PQ * HID, 128), jnp.float32)
    pW0b = jnp.zeros((128, PQ * proj_ch), jnp.float32)
    pW1b = jnp.zeros((PQ * proj_ch, PQ * out_ch), jnp.float32)
    for p in range(PQ):
        W0 = W0.at[96 + 2 * p:98 + 2 * p, HID * p:HID * (p + 1)].set(kW0[0:2])
        W0 = W0.at[102 + 2 * p:104 + 2 * p,
                   HID * p:HID * (p + 1)].set(kW0[2:4])
        W1 = W1.at[HID * p:HID * (p + 1),
                   IN_CH * p:IN_CH * (p + 1)].set(kW1)
        pW0b = pW0b.at[IN_CH * p:IN_CH * (p + 1),
                       proj_ch * p:proj_ch * (p + 1)].set(pW0)
        pW1b = pW1b.at[proj_ch * p:proj_ch * (p + 1),
                       out_ch * p:out_ch * (p + 1)].set(pW1)
    b0 = jnp.tile(kb0, PQ).reshape(1, -1)
    b1 = jnp.concatenate([jnp.tile(kb1, PQ),
                          jnp.zeros(128 - PQ * IN_CH)]).reshape(1, -1)
    pb0 = jnp.tile(pb0, PQ).reshape(1, -1)
    pb1 = jnp.tile(pb1, PQ).reshape(1, -1)

    full = lambda b: (0, 0)
    out = pl.pallas_call(
        _tc_body,
        grid=(nb,),
        in_specs=[
            pl.BlockSpec((K, BQ // PQ, 128), lambda b: (0, b, 0)),
            pl.BlockSpec(W0.shape, full),
            pl.BlockSpec(b0.shape, full),
            pl.BlockSpec(W1.shape, full),
            pl.BlockSpec(b1.shape, full),
            pl.BlockSpec(pW0b.shape, full),
            pl.BlockSpec(pb0.shape, full),
            pl.BlockSpec(pW1b.shape, full),
            pl.BlockSpec(pb1.shape, full),
        ],
        out_specs=pl.BlockSpec((BQ // PQ, PQ * out_ch), lambda b: (b, 0)),
        out_shape=jax.ShapeDtypeStruct((F_ROWS, PQ * out_ch), jnp.float32),
    )(rows, W0, b0, W1, b1, pW0b, pb0, pW1b, pb1)
    return out.reshape(Q_PAD, out_ch)[None, :Q, :]


# speedup vs baseline: 1.8562x; 1.8562x over previous
import functools

import jax
import jax.numpy as jnp
from jax import lax
from jax.experimental import pallas as pl
from jax.experimental.pallas import tpu as pltpu
from jax.experimental.pallas import tpu_sc as plsc

K_NB = 16
PQ = 3
BQ = 384
NW = 32
CQ = 48
IN_CH = 32
HID = 64
Q_PAD = 52224
F_ROWS = Q_PAD // PQ
N_CH = Q_PAD // (NW * CQ)


def _sc_chunk(ci, wid, idx_hbm, qc_hbm, rows_hbm, table_v, ltc_v, rbuf,
              cur, nxt, sem_i, sem_w):
    idx_v, qc_v = cur
    idx_n, qc_n = nxt
    qbase = wid * (N_CH * CQ) + ci * CQ
    lanes = lax.broadcasted_iota(jnp.int32, (16,), 0)

    @pl.when(ci + 1 < N_CH)
    def _prefetch():
        nq = qbase + CQ
        pltpu.async_copy(idx_hbm.at[pl.ds(nq * K_NB, CQ * K_NB)], idx_n,
                         sem_i)
        pltpu.async_copy(qc_hbm.at[pl.ds(nq * 2, CQ * 2)], qc_n, sem_i)

    @pl.when(ci >= 1)
    def _pf_wait():
        pltpu.make_async_copy(idx_hbm.at[pl.ds(0, CQ * K_NB)], idx_v,
                              sem_i).wait()
        pltpu.make_async_copy(qc_hbm.at[pl.ds(0, CQ * 2)], qc_v,
                              sem_i).wait()

    @pl.when(ci >= 1)
    def _wb_wait():
        pltpu.make_async_copy(rows_hbm.at[:, pl.ds(0, CQ // PQ), :], rbuf,
                              sem_w).wait()

    def regroup(g, c2):
        qloc = g * 16 + lanes
        rrow = (qloc * 21846) >> 16
        lbase = 2 * (qloc - 3 * rrow)
        qcx = plsc.load_gather(qc_v, [qloc * 2])
        qcy = plsc.load_gather(qc_v, [qloc * 2 + 1])

        def regroup_k(k, c3):
            krow = jnp.full((16,), 0, jnp.int32) + k
            vec = plsc.load_gather(idx_v, [qloc * K_NB + k])
            yx = plsc.load_gather(ltc_v, [vec * 2])
            yy = plsc.load_gather(ltc_v, [vec * 2 + 1])
            plsc.store_scatter(rbuf, [krow, rrow, 96 + lbase], yx)
            plsc.store_scatter(rbuf, [krow, rrow, 97 + lbase], yy)
            plsc.store_scatter(rbuf, [krow, rrow, 102 + lbase], qcx)
            plsc.store_scatter(rbuf, [krow, rrow, 103 + lbase], qcy)
            return c3

        return lax.fori_loop(0, K_NB, regroup_k, c2)

    lax.fori_loop(0, CQ // 16, regroup, 0)

    for h in (0, 1):

        def fill_q(qq, c2):
            q = h * (CQ // 2) + qq
            r = (q * 21846) >> 16
            lane0 = 32 * (q - 3 * r)
            kvec = idx_v[pl.ds(q * K_NB, K_NB)] * 16
            for k in range(K_NB):
                w = plsc.load_gather(table_v, [kvec[k] + lanes])
                lo = plsc.bitcast(w << 16, jnp.float32)
                hi = plsc.bitcast(w & jnp.int32(-65536), jnp.float32)
                rbuf[k, r, pl.ds(lane0, 16)] = lo
                rbuf[k, r, pl.ds(lane0 + 16, 16)] = hi
            return c2

        lax.fori_loop(0, CQ // 2, fill_q, 0)
        nr2 = CQ // PQ // 2
        pltpu.async_copy(
            rbuf.at[:, pl.ds(h * nr2, nr2), :],
            rows_hbm.at[:, pl.ds(qbase // PQ + h * nr2, nr2), :], sem_w)


def _sc_body(idx_hbm, table_hbm, ltc_hbm, qc_hbm, rows_hbm,
             idx0, qc0, idx1, qc1, rbuf, table_v, ltc_v,
             sem_i, sem_w):
    cid = lax.axis_index("c")
    sid = lax.axis_index("s")
    wid = sid * 2 + cid
    qbase0 = wid * (N_CH * CQ)
    lanes = lax.broadcasted_iota(jnp.int32, (16,), 0)
    zeros16 = jnp.zeros((16,), jnp.float32)

    pltpu.sync_copy(ltc_hbm, ltc_v)
    pltpu.sync_copy(table_hbm, table_v)

    def zinit(k, c2):
        krow = jnp.full((16,), 0, jnp.int32) + k

        def zinit_r(r, c3):
            rrow = jnp.full((16,), 0, jnp.int32) + r
            plsc.store_scatter(rbuf, [krow, rrow, 108 + lanes], zeros16)
            plsc.store_scatter(rbuf, [krow, rrow, 112 + lanes], zeros16)
            return c3

        return lax.fori_loop(0, CQ // PQ, zinit_r, c2)

    lax.fori_loop(0, K_NB, zinit, 0)

    pltpu.sync_copy(idx_hbm.at[pl.ds(qbase0 * K_NB, CQ * K_NB)], idx0)
    pltpu.sync_copy(qc_hbm.at[pl.ds(qbase0 * 2, CQ * 2)], qc0)

    set0 = (idx0, qc0)
    set1 = (idx1, qc1)

    def pair_body(i, carry):
        _sc_chunk(2 * i, wid, idx_hbm, qc_hbm, rows_hbm, table_v, ltc_v,
                  rbuf, set0, set1, sem_i, sem_w)
        _sc_chunk(2 * i + 1, wid, idx_hbm, qc_hbm, rows_hbm, table_v, ltc_v,
                  rbuf, set1, set0, sem_i, sem_w)
        return carry

    lax.fori_loop(0, N_CH // 2, pair_body, 0)
    pltpu.make_async_copy(rows_hbm.at[:, pl.ds(0, CQ // PQ), :], rbuf,
                          sem_w).wait()


def _sc_gather(idx_pad, table, ltc_flat, qc_flat):
    mesh = plsc.VectorSubcoreMesh(core_axis_name="c", subcore_axis_name="s")
    dbuf = lambda: [
        pltpu.VMEM((CQ * K_NB,), jnp.int32),
        pltpu.VMEM((CQ * 2,), jnp.float32),
    ]
    return pl.kernel(
        _sc_body,
        out_type=jax.ShapeDtypeStruct((K_NB, F_ROWS, 128), jnp.float32),
        mesh=mesh,
        scratch_types=dbuf() + dbuf() + [
            pltpu.VMEM((K_NB, CQ // PQ, 128), jnp.float32),
            pltpu.VMEM((table.shape[0],), jnp.int32),
            pltpu.VMEM((ltc_flat.shape[0],), jnp.float32),
            pltpu.SemaphoreType.DMA,
            pltpu.SemaphoreType.DMA,
        ],
        compiler_params=pltpu.CompilerParams(needs_layout_passes=False,
                                             use_tc_tiling_on_sc=False),
    )(idx_pad, table, ltc_flat, qc_flat)


def _dot(a, b):
    return jax.lax.dot_general(a, b, (((1,), (0,)), ((), ())),
                               preferred_element_type=jnp.float32)


def _tc_body(fr_ref, W0_ref, b0_ref, W1_ref, b1_ref, pW0_ref, pb0_ref,
             pW1_ref, pb1_ref, out_ref):
    nr = BQ // PQ
    fr = fr_ref[...]
    fr2 = fr.reshape(K_NB * nr, 128)
    h = jax.nn.gelu(_dot(fr2, W0_ref[...]) + b0_ref[...])
    kern = (_dot(h, W1_ref[...]) + b1_ref[...]).reshape(K_NB, nr, 128)
    dec = jnp.sum(kern * fr, axis=0)
    h2 = jax.nn.gelu(_dot(dec, pW0_ref[...]) + pb0_ref[...])
    out_ref[...] = _dot(h2, pW1_ref[...]) + pb1_ref[...]


def kernel(latent_tokens_coord, rndata, query_coord, nbr_index, nbr_row_splits,
           kW0, kb0, kW1, kb1, pW0, pb0, pW1, pb1):
    Q = query_coord.shape[0]
    K = nbr_index.shape[0] // Q
    assert K == K_NB
    out_ch = pW1.shape[1]
    proj_ch = pW0.shape[1]
    nb = Q_PAD // BQ

    idx_pad = jnp.pad(nbr_index, (0, (Q_PAD - Q) * K))
    qc_flat = jnp.pad(query_coord, ((0, Q_PAD - Q), (0, 0))).reshape(-1)
    rb = jax.lax.bitcast_convert_type(
        rndata[0].astype(jnp.bfloat16), jnp.uint16).astype(jnp.uint32)
    tword = (rb[:, :IN_CH // 2] | (rb[:, IN_CH // 2:] << 16))
    table = jax.lax.bitcast_convert_type(tword, jnp.int32).reshape(-1)
    rows = _sc_gather(idx_pad, table, latent_tokens_coord.reshape(-1),
                      qc_flat)

    W0 = jnp.zeros((128, PQ * HID), jnp.float32)
    W1 = jnp.zeros((PQ * HID, 128), jnp.float32)
    pW0b = jnp.zeros((128, PQ * proj_ch), jnp.float32)
    pW1b = jnp.zeros((PQ * proj_ch, PQ * out_ch), jnp.float32)
    for p in range(PQ):
        W0 = W0.at[96 + 2 * p:98 + 2 * p, HID * p:HID * (p + 1)].set(kW0[0:2])
        W0 = W0.at[102 + 2 * p:104 + 2 * p,
                   HID * p:HID * (p + 1)].set(kW0[2:4])
        W1 = W1.at[HID * p:HID * (p + 1),
                   IN_CH * p:IN_CH * (p + 1)].set(kW1)
        pW0b = pW0b.at[IN_CH * p:IN_CH * (p + 1),
                       proj_ch * p:proj_ch * (p + 1)].set(pW0)
        pW1b = pW1b.at[proj_ch * p:proj_ch * (p + 1),
                       out_ch * p:out_ch * (p + 1)].set(pW1)
    b0 = jnp.tile(kb0, PQ).reshape(1, -1)
    b1 = jnp.concatenate([jnp.tile(kb1, PQ),
                          jnp.zeros(128 - PQ * IN_CH)]).reshape(1, -1)
    pb0 = jnp.tile(pb0, PQ).reshape(1, -1)
    pb1 = jnp.tile(pb1, PQ).reshape(1, -1)

    full = lambda b: (0, 0)
    out = pl.pallas_call(
        _tc_body,
        grid=(nb,),
        in_specs=[
            pl.BlockSpec((K, BQ // PQ, 128), lambda b: (0, b, 0)),
            pl.BlockSpec(W0.shape, full),
            pl.BlockSpec(b0.shape, full),
            pl.BlockSpec(W1.shape, full),
            pl.BlockSpec(b1.shape, full),
            pl.BlockSpec(pW0b.shape, full),
            pl.BlockSpec(pb0.shape, full),
            pl.BlockSpec(pW1b.shape, full),
            pl.BlockSpec(pb1.shape, full),
        ],
        out_specs=pl.BlockSpec((BQ // PQ, PQ * out_ch), lambda b: (b, 0)),
        out_shape=jax.ShapeDtypeStruct((F_ROWS, PQ * out_ch), jnp.float32),
    )(rows, W0, b0, W1, b1, pW0b, pb0, pW1b, pb1)
    return out.reshape(Q_PAD, out_ch)[None, :Q, :]


# speedup vs baseline: 1.8896x; 1.0180x over previous
import functools

import jax
import jax.numpy as jnp
from jax import lax
from jax.experimental import pallas as pl
from jax.experimental.pallas import tpu as pltpu
from jax.experimental.pallas import tpu_sc as plsc

K_NB = 16
PQ = 3
BQ = 384
NW = 32
CQ = 48
IN_CH = 32
HID = 64
Q_PAD = 52224
F_ROWS = Q_PAD // PQ
N_CH = Q_PAD // (NW * CQ)


def _sc_chunk(ci, wid, idx_hbm, qc_hbm, rows_hbm, table_v, ltc_v, rbuf,
              cur, nxt, sem_i, sem_w):
    idx_v, qc_v = cur
    idx_n, qc_n = nxt
    qbase = wid * (N_CH * CQ) + ci * CQ
    n_q = qc_hbm.shape[0] // 2
    lanes = lax.broadcasted_iota(jnp.int32, (16,), 0)

    @pl.when(ci + 1 < N_CH)
    def _prefetch():
        nq = jnp.minimum(qbase + CQ, n_q - CQ)
        pltpu.async_copy(idx_hbm.at[pl.ds(nq * K_NB, CQ * K_NB)], idx_n,
                         sem_i)
        pltpu.async_copy(qc_hbm.at[pl.ds(nq * 2, CQ * 2)], qc_n, sem_i)

    @pl.when(ci >= 1)
    def _pf_wait():
        pltpu.make_async_copy(idx_hbm.at[pl.ds(0, CQ * K_NB)], idx_v,
                              sem_i).wait()
        pltpu.make_async_copy(qc_hbm.at[pl.ds(0, CQ * 2)], qc_v,
                              sem_i).wait()

    @pl.when(ci >= 1)
    def _wb_wait():
        pltpu.make_async_copy(rows_hbm.at[:, pl.ds(0, CQ // PQ), :], rbuf,
                              sem_w).wait()

    def regroup(g, c2):
        qloc = g * 16 + lanes
        rrow = (qloc * 21846) >> 16
        lbase = 2 * (qloc - 3 * rrow)
        qcx = plsc.load_gather(qc_v, [qloc * 2])
        qcy = plsc.load_gather(qc_v, [qloc * 2 + 1])

        def regroup_k(k, c3):
            krow = jnp.full((16,), 0, jnp.int32) + k
            vec = plsc.load_gather(idx_v, [qloc * K_NB + k])
            yx = plsc.load_gather(ltc_v, [vec * 2])
            yy = plsc.load_gather(ltc_v, [vec * 2 + 1])
            plsc.store_scatter(rbuf, [krow, rrow, 96 + lbase], yx)
            plsc.store_scatter(rbuf, [krow, rrow, 97 + lbase], yy)
            plsc.store_scatter(rbuf, [krow, rrow, 102 + lbase], qcx)
            plsc.store_scatter(rbuf, [krow, rrow, 103 + lbase], qcy)
            return c3

        return lax.fori_loop(0, K_NB, regroup_k, c2)

    lax.fori_loop(0, CQ // 16, regroup, 0)

    for h in (0, 1):

        def fill_q(qq, c2):
            q = h * (CQ // 2) + qq
            r = (q * 21846) >> 16
            lane0 = 32 * (q - 3 * r)
            kvec = idx_v[pl.ds(q * K_NB, K_NB)] * 16
            for k in range(K_NB):
                w = plsc.load_gather(table_v, [kvec[k] + lanes])
                lo = plsc.bitcast(w << 16, jnp.float32)
                hi = plsc.bitcast(w & jnp.int32(-65536), jnp.float32)
                rbuf[k, r, pl.ds(lane0, 16)] = lo
                rbuf[k, r, pl.ds(lane0 + 16, 16)] = hi
            return c2

        lax.fori_loop(0, CQ // 2, fill_q, 0)
        nr2 = CQ // PQ // 2
        pltpu.async_copy(
            rbuf.at[:, pl.ds(h * nr2, nr2), :],
            rows_hbm.at[:, pl.ds(qbase // PQ + h * nr2, nr2), :], sem_w)


def _sc_body(idx_hbm, table_hbm, ltc_hbm, qc_hbm, rows_hbm,
             idx0, qc0, idx1, qc1, rbuf, table_v, ltc_v,
             sem_i, sem_w):
    cid = lax.axis_index("c")
    sid = lax.axis_index("s")
    wid = sid * 2 + cid
    qbase0 = wid * (N_CH * CQ)
    lanes = lax.broadcasted_iota(jnp.int32, (16,), 0)
    zeros16 = jnp.zeros((16,), jnp.float32)

    pltpu.sync_copy(ltc_hbm, ltc_v)
    pltpu.sync_copy(table_hbm, table_v)

    def zinit(k, c2):
        krow = jnp.full((16,), 0, jnp.int32) + k

        def zinit_r(r, c3):
            rrow = jnp.full((16,), 0, jnp.int32) + r
            plsc.store_scatter(rbuf, [krow, rrow, 108 + lanes], zeros16)
            plsc.store_scatter(rbuf, [krow, rrow, 112 + lanes], zeros16)
            return c3

        return lax.fori_loop(0, CQ // PQ, zinit_r, c2)

    lax.fori_loop(0, K_NB, zinit, 0)

    qb0 = jnp.minimum(qbase0, qc_hbm.shape[0] // 2 - CQ)
    pltpu.sync_copy(idx_hbm.at[pl.ds(qb0 * K_NB, CQ * K_NB)], idx0)
    pltpu.sync_copy(qc_hbm.at[pl.ds(qb0 * 2, CQ * 2)], qc0)

    set0 = (idx0, qc0)
    set1 = (idx1, qc1)

    def pair_body(i, carry):
        _sc_chunk(2 * i, wid, idx_hbm, qc_hbm, rows_hbm, table_v, ltc_v,
                  rbuf, set0, set1, sem_i, sem_w)
        _sc_chunk(2 * i + 1, wid, idx_hbm, qc_hbm, rows_hbm, table_v, ltc_v,
                  rbuf, set1, set0, sem_i, sem_w)
        return carry

    lax.fori_loop(0, N_CH // 2, pair_body, 0)
    pltpu.make_async_copy(rows_hbm.at[:, pl.ds(0, CQ // PQ), :], rbuf,
                          sem_w).wait()


def _sc_gather(idx_pad, table, ltc_flat, qc_flat):
    mesh = plsc.VectorSubcoreMesh(core_axis_name="c", subcore_axis_name="s")
    dbuf = lambda: [
        pltpu.VMEM((CQ * K_NB,), jnp.int32),
        pltpu.VMEM((CQ * 2,), jnp.float32),
    ]
    return pl.kernel(
        _sc_body,
        out_type=jax.ShapeDtypeStruct((K_NB, F_ROWS, 128), jnp.float32),
        mesh=mesh,
        scratch_types=dbuf() + dbuf() + [
            pltpu.VMEM((K_NB, CQ // PQ, 128), jnp.float32),
            pltpu.VMEM((table.shape[0],), jnp.int32),
            pltpu.VMEM((ltc_flat.shape[0],), jnp.float32),
            pltpu.SemaphoreType.DMA,
            pltpu.SemaphoreType.DMA,
        ],
        compiler_params=pltpu.CompilerParams(needs_layout_passes=False,
                                             use_tc_tiling_on_sc=False),
    )(idx_pad, table, ltc_flat, qc_flat)


def _dot(a, b):
    return jax.lax.dot_general(a, b, (((1,), (0,)), ((), ())),
                               preferred_element_type=jnp.float32)


def _tc_body(fr_ref, W0_ref, b0_ref, W1_ref, b1_ref, pW0_ref, pb0_ref,
             pW1_ref, pb1_ref, out_ref):
    nr = BQ // PQ
    fr = fr_ref[...]
    fr2 = fr.reshape(K_NB * nr, 128).astype(jnp.bfloat16)
    h = jax.nn.gelu(_dot(fr2, W0_ref[...]) + b0_ref[...])
    kern = (_dot(h.astype(jnp.bfloat16), W1_ref[...]) + b1_ref[...])
    kern = kern.reshape(K_NB, nr, 128)
    dec = jnp.sum(kern * fr, axis=0)
    h2 = jax.nn.gelu(_dot(dec.astype(jnp.bfloat16), pW0_ref[...])
                     + pb0_ref[...])
    out_ref[...] = (_dot(h2.astype(jnp.bfloat16), pW1_ref[...])
                    + pb1_ref[...])


def kernel(latent_tokens_coord, rndata, query_coord, nbr_index, nbr_row_splits,
           kW0, kb0, kW1, kb1, pW0, pb0, pW1, pb1):
    Q = query_coord.shape[0]
    K = nbr_index.shape[0] // Q
    assert K == K_NB
    out_ch = pW1.shape[1]
    proj_ch = pW0.shape[1]
    nb = Q_PAD // BQ

    rb = jax.lax.bitcast_convert_type(
        rndata[0].astype(jnp.bfloat16), jnp.uint16).astype(jnp.uint32)
    tword = (rb[:, :IN_CH // 2] | (rb[:, IN_CH // 2:] << 16))
    table = jax.lax.bitcast_convert_type(tword, jnp.int32).reshape(-1)
    rows = _sc_gather(nbr_index, table, latent_tokens_coord.reshape(-1),
                      query_coord.reshape(-1))

    W0 = jnp.zeros((128, PQ * HID), jnp.float32)
    W1 = jnp.zeros((PQ * HID, 128), jnp.float32)
    pW0b = jnp.zeros((128, PQ * proj_ch), jnp.float32)
    pW1b = jnp.zeros((PQ * proj_ch, PQ * out_ch), jnp.float32)
    for p in range(PQ):
        W0 = W0.at[96 + 2 * p:98 + 2 * p, HID * p:HID * (p + 1)].set(kW0[0:2])
        W0 = W0.at[102 + 2 * p:104 + 2 * p,
                   HID * p:HID * (p + 1)].set(kW0[2:4])
        W1 = W1.at[HID * p:HID * (p + 1),
                   IN_CH * p:IN_CH * (p + 1)].set(kW1)
        pW0b = pW0b.at[IN_CH * p:IN_CH * (p + 1),
                       proj_ch * p:proj_ch * (p + 1)].set(pW0)
        pW1b = pW1b.at[proj_ch * p:proj_ch * (p + 1),
                       out_ch * p:out_ch * (p + 1)].set(pW1)
    b0 = jnp.tile(kb0, PQ).reshape(1, -1)
    b1 = jnp.concatenate([jnp.tile(kb1, PQ),
                          jnp.zeros(128 - PQ * IN_CH)]).reshape(1, -1)
    pb0 = jnp.tile(pb0, PQ).reshape(1, -1)
    pb1 = jnp.tile(pb1, PQ).reshape(1, -1)

    full = lambda b: (0, 0)
    out = pl.pallas_call(
        _tc_body,
        grid=(nb,),
        in_specs=[
            pl.BlockSpec((K, BQ // PQ, 128), lambda b: (0, b, 0)),
            pl.BlockSpec(W0.shape, full),
            pl.BlockSpec(b0.shape, full),
            pl.BlockSpec(W1.shape, full),
            pl.BlockSpec(b1.shape, full),
            pl.BlockSpec(pW0b.shape, full),
            pl.BlockSpec(pb0.shape, full),
            pl.BlockSpec(pW1b.shape, full),
            pl.BlockSpec(pb1.shape, full),
        ],
        out_specs=pl.BlockSpec((BQ // PQ, PQ * out_ch), lambda b: (b, 0)),
        out_shape=jax.ShapeDtypeStruct((F_ROWS, PQ * out_ch), jnp.float32),
    )(rows, W0.astype(jnp.bfloat16), b0, W1.astype(jnp.bfloat16), b1,
      pW0b.astype(jnp.bfloat16), pb0, pW1b.astype(jnp.bfloat16), pb1)
    return out.reshape(Q_PAD, out_ch)[None, :Q, :]
